# Initial kernel scaffold; baseline (speedup 1.0000x reference)
import functools

import jax
import jax.numpy as jnp
from jax import lax
from jax.experimental import pallas as pl
from jax.experimental.pallas import tpu as pltpu
from jax.experimental.pallas import tpu_sc as plsc

N = 10000
D = 128
N_PAD = 10240
NC = 2
NS = 16
TILES = NC * NS
CHUNK = 128
CPT = 80
E_PAD = TILES * CPT * CHUNK
ROWS_PER_TILE = N_PAD // NS
BLK = 512
GRID = N_PAD // BLK

_mesh = plsc.VectorSubcoreMesh(core_axis_name="c", subcore_axis_name="s")


def _zero_shared_slice(zbuf, acc_sh, sid, width):
    def zrow(i, _):
        for j in range(width // 16):
            zbuf[i, pl.ds(j * 16, 16)] = jnp.zeros((16,), jnp.float32)
        return 0
    lax.fori_loop(0, CHUNK, zrow, 0)
    for k in range(ROWS_PER_TILE // CHUNK):
        pltpu.sync_copy(zbuf, acc_sh.at[pl.ds(sid * ROWS_PER_TILE + k * CHUNK, CHUNK)])


@functools.partial(
    pl.kernel,
    out_type=jax.ShapeDtypeStruct((NC, N_PAD, 16), jnp.float32),
    mesh=_mesh,
    scratch_types=[
        pltpu.VMEM((CPT, CHUNK), jnp.int32),
        pltpu.VMEM((CHUNK, 16), jnp.float32),
        pltpu.VMEM_SHARED((N_PAD, 16), jnp.float32),
    ],
)
def _deg_kernel(dst_hbm, out_hbm, dst_v, ones_v, acc_sh):
    cid = lax.axis_index("c")
    sid = lax.axis_index("s")
    wid = cid * NS + sid
    _zero_shared_slice(ones_v, acc_sh, sid, 16)

    def orow(i, _):
        ones_v[i, pl.ds(0, 16)] = jnp.ones((16,), jnp.float32)
        return 0
    lax.fori_loop(0, CHUNK, orow, 0)
    pltpu.sync_copy(dst_hbm.at[wid], dst_v)
    plsc.subcore_barrier()

    def step(j, _):
        pltpu.sync_copy(ones_v, acc_sh.at[dst_v.at[j]], add=True)
        return 0
    lax.fori_loop(0, CPT, step, 0)
    plsc.subcore_barrier()
    pltpu.sync_copy(acc_sh.at[pl.ds(sid * ROWS_PER_TILE, ROWS_PER_TILE)],
                    out_hbm.at[cid, pl.ds(sid * ROWS_PER_TILE, ROWS_PER_TILE)])


@functools.partial(
    pl.kernel,
    out_type=jax.ShapeDtypeStruct((NC, N_PAD, D), jnp.float32),
    mesh=_mesh,
    scratch_types=[
        pltpu.VMEM((CPT, CHUNK), jnp.int32),
        pltpu.VMEM((CPT, CHUNK), jnp.int32),
        pltpu.VMEM((CHUNK, D), jnp.float32),
        pltpu.VMEM((CHUNK, D), jnp.float32),
        pltpu.VMEM_SHARED((N_PAD, D), jnp.float32),
        pltpu.SemaphoreType.DMA,
        pltpu.SemaphoreType.DMA,
    ],
)
def _prop_kernel(table_hbm, src_hbm, dst_hbm, out_hbm,
                 src_v, dst_v, rows0, rows1, acc_sh, sem0, sem1):
    cid = lax.axis_index("c")
    sid = lax.axis_index("s")
    wid = cid * NS + sid
    _zero_shared_slice(rows0, acc_sh, sid, D)
    pltpu.sync_copy(src_hbm.at[wid], src_v)
    pltpu.sync_copy(dst_hbm.at[wid], dst_v)
    plsc.subcore_barrier()

    def pair(g, _):
        j0 = 2 * g
        c0 = pltpu.async_copy(table_hbm.at[src_v.at[j0]], rows0, sem0)
        c1 = pltpu.async_copy(table_hbm.at[src_v.at[j0 + 1]], rows1, sem1)
        c0.wait()
        pltpu.sync_copy(rows0, acc_sh.at[dst_v.at[j0]], add=True)
        c1.wait()
        pltpu.sync_copy(rows1, acc_sh.at[dst_v.at[j0 + 1]], add=True)
        return 0
    lax.fori_loop(0, CPT // 2, pair, 0)
    plsc.subcore_barrier()
    pltpu.sync_copy(acc_sh.at[pl.ds(sid * ROWS_PER_TILE, ROWS_PER_TILE)],
                    out_hbm.at[cid, pl.ds(sid * ROWS_PER_TILE, ROWS_PER_TILE)])


def _dis_block(degp, i):
    deg = degp[0, :, 0:1] + degp[1, :, 0:1]
    dis = lax.rsqrt(1.0 + deg)
    rows = lax.broadcasted_iota(jnp.int32, (BLK, 1), 0) + i * BLK
    return dis, rows < N


def _stage1_body(x_ref, w_ref, degp_ref, o_ref):
    i = pl.program_id(0)
    dis, valid = _dis_block(degp_ref, i)
    y = jnp.dot(x_ref[...], w_ref[...], preferred_element_type=jnp.float32)
    o_ref[...] = jnp.where(valid, y * dis, 0.0)


def _stage2_body(g_ref, t1_ref, degp_ref, b_ref, o_ref):
    i = pl.program_id(0)
    dis, valid = _dis_block(degp_ref, i)
    s = (g_ref[0] + g_ref[1] + t1_ref[...]) * dis + b_ref[...]
    h = jnp.maximum(s, 0.0)
    o_ref[...] = jnp.where(valid, h * dis, 0.0)


def _stage3_body(g_ref, t2_ref, degp_ref, w_ref, b_ref, o_ref):
    i = pl.program_id(0)
    dis, _ = _dis_block(degp_ref, i)
    p = (g_ref[0] + g_ref[1] + t2_ref[...]) * dis
    o_ref[...] = jnp.dot(p, w_ref[...], preferred_element_type=jnp.float32) + b_ref[...]


_row_spec = pl.BlockSpec((BLK, D), lambda i: (i, 0))
_deg_spec = pl.BlockSpec((2, BLK, 16), lambda i: (0, i, 0))
_g_spec = pl.BlockSpec((2, BLK, D), lambda i: (0, i, 0))
_w_spec = pl.BlockSpec((D, D), lambda i: (0, 0))
_b_spec = pl.BlockSpec((1, D), lambda i: (0, 0))
_out = jax.ShapeDtypeStruct((N_PAD, D), jnp.float32)

_stage1 = pl.pallas_call(
    _stage1_body, grid=(GRID,), out_shape=_out,
    in_specs=[_row_spec, _w_spec, _deg_spec], out_specs=_row_spec)
_stage2 = pl.pallas_call(
    _stage2_body, grid=(GRID,), out_shape=_out,
    in_specs=[_g_spec, _row_spec, _deg_spec, _b_spec], out_specs=_row_spec)
_stage3 = pl.pallas_call(
    _stage3_body, grid=(GRID,), out_shape=_out,
    in_specs=[_g_spec, _row_spec, _deg_spec, _w_spec, _b_spec], out_specs=_row_spec)


def kernel(x, edge_index, W1, b1, W_mu, b_mu, W_ls, b_ls):
    E = edge_index.shape[1]
    src = edge_index[0].astype(jnp.int32)
    dst = edge_index[1].astype(jnp.int32)
    pad = jnp.full((E_PAD - E,), N, dtype=jnp.int32)
    srcp = jnp.concatenate([src, pad]).reshape(TILES, CPT, CHUNK)
    dstp = jnp.concatenate([dst, pad]).reshape(TILES, CPT, CHUNK)
    x_pad = jnp.pad(x, ((0, N_PAD - N), (0, 0)))

    degp = _deg_kernel(dstp)
    t1 = _stage1(x_pad, W1, degp)
    g1 = _prop_kernel(t1, srcp, dstp)
    t2 = _stage2(g1, t1, degp, b1.reshape(1, D))
    g2 = _prop_kernel(t2, srcp, dstp)
    wcat = jnp.concatenate([W_mu, W_ls], axis=1)
    bcat = jnp.concatenate([b_mu, b_ls]).reshape(1, D)
    out = _stage3(g2, t2, degp, wcat, bcat)
    return out[:N, :64], out[:N, 64:]


# speedup vs baseline: 10.6482x; 10.6482x over previous
import functools

import jax
import jax.numpy as jnp
from jax import lax
from jax.experimental import pallas as pl
from jax.experimental.pallas import tpu as pltpu
from jax.experimental.pallas import tpu_sc as plsc

N = 10000
D = 128
N_PAD = 10240
NC = 2
NS = 16
TILES = NC * NS
CHUNK = 128
CPT = 80
E_PAD = TILES * CPT * CHUNK
ROWS_PER_TILE = N_PAD // NS
BLK = 512
GRID = N_PAD // BLK

_mesh = plsc.VectorSubcoreMesh(core_axis_name="c", subcore_axis_name="s")


def _zero_shared_slice(zbuf, acc_sh, sid, width):
    def zrow(i, _):
        for j in range(width // 16):
            zbuf[i, pl.ds(j * 16, 16)] = jnp.zeros((16,), jnp.float32)
        return 0
    lax.fori_loop(0, CHUNK, zrow, 0)
    for k in range(ROWS_PER_TILE // CHUNK):
        pltpu.sync_copy(zbuf, acc_sh.at[pl.ds(sid * ROWS_PER_TILE + k * CHUNK, CHUNK)])


@functools.partial(
    pl.kernel,
    out_type=jax.ShapeDtypeStruct((NC, N_PAD, 16), jnp.float32),
    mesh=_mesh,
    scratch_types=[
        pltpu.VMEM((CPT, CHUNK), jnp.int32),
        pltpu.VMEM((CHUNK, 16), jnp.float32),
        pltpu.VMEM_SHARED((N_PAD, 16), jnp.float32),
    ],
)
def _deg_kernel(dst_hbm, out_hbm, dst_v, ones_v, acc_sh):
    cid = lax.axis_index("c")
    sid = lax.axis_index("s")
    wid = cid * NS + sid
    _zero_shared_slice(ones_v, acc_sh, sid, 16)

    def orow(i, _):
        ones_v[i, pl.ds(0, 16)] = jnp.ones((16,), jnp.float32)
        return 0
    lax.fori_loop(0, CHUNK, orow, 0)
    pltpu.sync_copy(dst_hbm.at[wid], dst_v)
    plsc.subcore_barrier()

    def step(j, _):
        pltpu.sync_copy(ones_v, acc_sh.at[dst_v.at[j]], add=True)
        return 0
    lax.fori_loop(0, CPT, step, 0)
    plsc.subcore_barrier()
    pltpu.sync_copy(acc_sh.at[pl.ds(sid * ROWS_PER_TILE, ROWS_PER_TILE)],
                    out_hbm.at[cid, pl.ds(sid * ROWS_PER_TILE, ROWS_PER_TILE)])


@functools.partial(
    pl.kernel,
    out_type=jax.ShapeDtypeStruct((NC, N_PAD, D), jnp.float32),
    mesh=_mesh,
    scratch_types=[
        pltpu.VMEM((CHUNK,), jnp.int32),
        pltpu.VMEM((CHUNK,), jnp.int32),
        pltpu.VMEM((CHUNK,), jnp.int32),
        pltpu.VMEM((CHUNK,), jnp.int32),
        pltpu.VMEM((CHUNK, D), jnp.float32),
        pltpu.VMEM((CHUNK, D), jnp.float32),
        pltpu.VMEM_SHARED((N_PAD, D), jnp.float32),
        pltpu.SemaphoreType.DMA,
        pltpu.SemaphoreType.DMA,
        pltpu.SemaphoreType.DMA,
        pltpu.SemaphoreType.DMA,
    ],
)
def _prop_kernel(table_hbm, src_hbm, dst_hbm, out_hbm,
                 sbuf0, sbuf1, dbuf0, dbuf1, rows0, rows1, acc_sh,
                 sem0, sem1, semi0, semi1):
    cid = lax.axis_index("c")
    sid = lax.axis_index("s")
    wid = cid * NS + sid
    _zero_shared_slice(rows0, acc_sh, sid, D)
    plsc.subcore_barrier()

    def pair(g, _):
        j0 = 2 * g
        ci0 = pltpu.async_copy(src_hbm.at[wid, j0], sbuf0, semi0)
        ci1 = pltpu.async_copy(src_hbm.at[wid, j0 + 1], sbuf1, semi1)
        pltpu.sync_copy(dst_hbm.at[wid, j0], dbuf0)
        pltpu.sync_copy(dst_hbm.at[wid, j0 + 1], dbuf1)
        ci0.wait()
        c0 = pltpu.async_copy(table_hbm.at[sbuf0], rows0, sem0)
        ci1.wait()
        c1 = pltpu.async_copy(table_hbm.at[sbuf1], rows1, sem1)
        c0.wait()
        pltpu.sync_copy(rows0, acc_sh.at[dbuf0], add=True)
        c1.wait()
        pltpu.sync_copy(rows1, acc_sh.at[dbuf1], add=True)
        return 0
    lax.fori_loop(0, CPT // 2, pair, 0)
    plsc.subcore_barrier()
    pltpu.sync_copy(acc_sh.at[pl.ds(sid * ROWS_PER_TILE, ROWS_PER_TILE)],
                    out_hbm.at[cid, pl.ds(sid * ROWS_PER_TILE, ROWS_PER_TILE)])


def _dis_block(degp, i):
    deg = degp[0, :, 0:1] + degp[1, :, 0:1]
    dis = lax.rsqrt(1.0 + deg)
    rows = lax.broadcasted_iota(jnp.int32, (BLK, 1), 0) + i * BLK
    return dis, rows < N


def _stage1_body(x_ref, w_ref, degp_ref, o_ref):
    i = pl.program_id(0)
    dis, valid = _dis_block(degp_ref, i)
    y = jnp.dot(x_ref[...], w_ref[...], preferred_element_type=jnp.float32)
    o_ref[...] = jnp.where(valid, y * dis, 0.0)


def _stage2_body(g_ref, t1_ref, degp_ref, b_ref, o_ref):
    i = pl.program_id(0)
    dis, valid = _dis_block(degp_ref, i)
    s = (g_ref[0] + g_ref[1] + t1_ref[...]) * dis + b_ref[...]
    h = jnp.maximum(s, 0.0)
    o_ref[...] = jnp.where(valid, h * dis, 0.0)


def _stage3_body(g_ref, t2_ref, degp_ref, w_ref, b_ref, o_ref):
    i = pl.program_id(0)
    dis, _ = _dis_block(degp_ref, i)
    p = (g_ref[0] + g_ref[1] + t2_ref[...]) * dis
    o_ref[...] = jnp.dot(p, w_ref[...], preferred_element_type=jnp.float32) + b_ref[...]


_row_spec = pl.BlockSpec((BLK, D), lambda i: (i, 0))
_deg_spec = pl.BlockSpec((2, BLK, 16), lambda i: (0, i, 0))
_g_spec = pl.BlockSpec((2, BLK, D), lambda i: (0, i, 0))
_w_spec = pl.BlockSpec((D, D), lambda i: (0, 0))
_b_spec = pl.BlockSpec((1, D), lambda i: (0, 0))
_out = jax.ShapeDtypeStruct((N_PAD, D), jnp.float32)

_stage1 = pl.pallas_call(
    _stage1_body, grid=(GRID,), out_shape=_out,
    in_specs=[_row_spec, _w_spec, _deg_spec], out_specs=_row_spec)
_stage2 = pl.pallas_call(
    _stage2_body, grid=(GRID,), out_shape=_out,
    in_specs=[_g_spec, _row_spec, _deg_spec, _b_spec], out_specs=_row_spec)
_stage3 = pl.pallas_call(
    _stage3_body, grid=(GRID,), out_shape=_out,
    in_specs=[_g_spec, _row_spec, _deg_spec, _w_spec, _b_spec], out_specs=_row_spec)


def kernel(x, edge_index, W1, b1, W_mu, b_mu, W_ls, b_ls):
    E = edge_index.shape[1]
    src = edge_index[0].astype(jnp.int32)
    dst = edge_index[1].astype(jnp.int32)
    pad = jnp.full((E_PAD - E,), N, dtype=jnp.int32)
    srcp = jnp.concatenate([src, pad]).reshape(TILES, CPT, CHUNK)
    dstp = jnp.concatenate([dst, pad]).reshape(TILES, CPT, CHUNK)
    x_pad = jnp.pad(x, ((0, N_PAD - N), (0, 0)))

    degp = _deg_kernel(dstp)
    t1 = _stage1(x_pad, W1, degp)
    g1 = _prop_kernel(t1, srcp, dstp)
    t2 = _stage2(g1, t1, degp, b1.reshape(1, D))
    g2 = _prop_kernel(t2, srcp, dstp)
    wcat = jnp.concatenate([W_mu, W_ls], axis=1)
    bcat = jnp.concatenate([b_mu, b_ls]).reshape(1, D)
    out = _stage3(g2, t2, degp, wcat, bcat)
    return out[:N, :64], out[:N, 64:]
